# baseline (speedup 1.0000x reference)
import functools

import jax
import jax.numpy as jnp
from jax import lax
from jax.experimental import pallas as pl
from jax.experimental.pallas import tpu as pltpu
from jax.experimental.pallas import tpu_sc as plsc

Q = 1024
D = 128
N = 100000
C = 1000
K = 50
TN = 2048
NT = 49
NP = NT * TN
BLK = 128
NB = NP // BLK
B_TOT = Q * K
NW = 32
B_PER_W = B_TOT // NW
CH = 400
QB = 128
CAND = K * BLK

NEG_INF = float("-inf")


def _mm_kernel(x_ref, db_ref, s_ref, bm_ref):
    i = pl.program_id(0)
    s = lax.dot_general(
        x_ref[...], db_ref[...], (((1,), (1,)), ((), ())),
        preferred_element_type=jnp.float32,
        precision=lax.Precision.DEFAULT,
    )
    col = i * TN + lax.broadcasted_iota(jnp.int32, (Q, TN), 1)
    s = jnp.where(col < N, s, NEG_INF)
    for b in range(TN // BLK):
        s_ref[b * Q:(b + 1) * Q, :] = s[:, b * BLK:(b + 1) * BLK]
    bm_ref[0] = jnp.concatenate(
        [jnp.max(s[:, b * BLK:(b + 1) * BLK], axis=1, keepdims=True)
         for b in range(TN // BLK)], axis=1)


_k1 = pl.pallas_call(
    _mm_kernel,
    grid=(NT,),
    in_specs=[
        pl.BlockSpec((Q, D), lambda i: (0, 0)),
        pl.BlockSpec((TN, D), lambda i: (i, 0)),
    ],
    out_specs=[
        pl.BlockSpec(((TN // BLK) * Q, BLK), lambda i: (i, 0)),
        pl.BlockSpec((1, Q, TN // BLK), lambda i: (i, 0, 0)),
    ],
    out_shape=[
        jax.ShapeDtypeStruct((Q * NB, BLK), jnp.float32),
        jax.ShapeDtypeStruct((NT, Q, TN // BLK), jnp.float32),
    ],
)


def _sel_kernel(bm_ref, sidx_ref, lidx_ref, bm_s):
    bm_s[...] = bm_ref[...]
    colio = lax.broadcasted_iota(jnp.int32, (Q, NB), 1)
    qrow = lax.broadcasted_iota(jnp.int32, (Q, 1), 0)
    kio = lax.broadcasted_iota(jnp.int32, (Q, K), 1)

    def body(r, _):
        bm = bm_s[...]
        m = jnp.max(bm, axis=1, keepdims=True)
        idx = jnp.min(jnp.where(bm == m, colio, NB), axis=1, keepdims=True)
        sel = kio == r
        lidx_ref[...] = jnp.where(sel, idx, lidx_ref[...])
        srow = (idx >> 4) * ((TN // BLK) * Q) + (idx & (TN // BLK - 1)) * Q + qrow
        sidx_ref[...] = jnp.where(sel, srow, sidx_ref[...])
        bm_s[...] = jnp.where(colio == idx, NEG_INF, bm)
        return 0

    lax.fori_loop(0, K, body, 0)


_k2 = pl.pallas_call(
    _sel_kernel,
    in_specs=[pl.BlockSpec((Q, NB), lambda: (0, 0))],
    out_specs=[
        pl.BlockSpec((Q, K), lambda: (0, 0)),
        pl.BlockSpec((Q, K), lambda: (0, 0)),
    ],
    out_shape=[
        jax.ShapeDtypeStruct((Q, K), jnp.int32),
        jax.ShapeDtypeStruct((Q, K), jnp.int32),
    ],
    scratch_shapes=[pltpu.VMEM((Q, NB), jnp.float32)],
)


def _gather_body(s_hbm, sidx_hbm, lab_hbm, lidx_hbm, outs_hbm, outl_hbm,
                 sidx_v, srows_v, lidx_v, lrows_v, sem1, sem2):
    wid = lax.axis_index("s") * 2 + lax.axis_index("c")
    base = wid * B_PER_W
    for c in range(B_PER_W // CH):
        off = base + c * CH
        pltpu.sync_copy(sidx_hbm.at[pl.ds(off, CH)], sidx_v)
        pltpu.sync_copy(lidx_hbm.at[pl.ds(off, CH)], lidx_v)
        cp1 = pltpu.async_copy(s_hbm.at[sidx_v], srows_v, sem1)
        cp2 = pltpu.async_copy(lab_hbm.at[lidx_v], lrows_v, sem2)
        cp1.wait()
        cp2.wait()
        pltpu.sync_copy(srows_v, outs_hbm.at[pl.ds(off, CH)])
        pltpu.sync_copy(lrows_v, outl_hbm.at[pl.ds(off, CH)])


_k3 = functools.partial(
    pl.kernel,
    out_type=(
        jax.ShapeDtypeStruct((B_TOT, BLK), jnp.float32),
        jax.ShapeDtypeStruct((B_TOT, BLK), jnp.int32),
    ),
    mesh=plsc.VectorSubcoreMesh(core_axis_name="c", subcore_axis_name="s"),
    scratch_types=[
        pltpu.VMEM((CH,), jnp.int32),
        pltpu.VMEM((CH, BLK), jnp.float32),
        pltpu.VMEM((CH,), jnp.int32),
        pltpu.VMEM((CH, BLK), jnp.int32),
        pltpu.SemaphoreType.DMA,
        pltpu.SemaphoreType.DMA,
    ],
)(_gather_body)


def _hist_kernel(cs_ref, cl_ref, out_ref, h_s):
    h_s[...] = jnp.zeros((QB, C), jnp.float32)
    cio = lax.broadcasted_iota(jnp.int32, (QB, C), 1)

    def body(r, _):
        s3 = cs_ref[...].reshape(QB, K, BLK)
        lab3 = cl_ref[...].reshape(QB, K, BLK)
        m = jnp.max(jnp.max(s3, axis=2, keepdims=True), axis=1, keepdims=True)
        hit = s3 == m
        lbl3 = jnp.max(jnp.max(jnp.where(hit, lab3, -1), axis=2, keepdims=True),
                       axis=1, keepdims=True)
        h_s[...] = h_s[...] + (cio == lbl3.reshape(QB, 1)).astype(jnp.float32)
        cs_ref[...] = jnp.where(hit & (lab3 == lbl3), NEG_INF, s3).reshape(QB * K, BLK)
        return 0

    lax.fori_loop(0, K, body, 0)
    out_ref[...] = h_s[...] * jnp.float32(1.0 / K)


_k4 = pl.pallas_call(
    _hist_kernel,
    grid=(Q // QB,),
    in_specs=[
        pl.BlockSpec((QB * K, BLK), lambda i: (i, 0)),
        pl.BlockSpec((QB * K, BLK), lambda i: (i, 0)),
    ],
    out_specs=pl.BlockSpec((QB, C), lambda i: (i, 0)),
    out_shape=jax.ShapeDtypeStruct((Q, C), jnp.float32),
    scratch_shapes=[
        pltpu.VMEM((QB, C), jnp.float32),
    ],
)


def kernel(x, database, aux_labels):
    s, bm3 = _k1(x, database)
    sidx, lidx = _k2(bm3.transpose(1, 0, 2).reshape(Q, NB))
    labtab = jnp.concatenate(
        [aux_labels, jnp.zeros((NP - N,), jnp.int32)]).reshape(NB, BLK)
    cand_s, cand_l = _k3(
        s, sidx.reshape(B_TOT), labtab, lidx.reshape(B_TOT))
    return _k4(cand_s, cand_l)


# speedup vs baseline: 3.2573x; 3.2573x over previous
import functools

import jax
import jax.numpy as jnp
from jax import lax
from jax.experimental import pallas as pl
from jax.experimental.pallas import tpu as pltpu
from jax.experimental.pallas import tpu_sc as plsc

Q = 1024
D = 128
N = 100000
C = 1000
K = 50
TN = 2048
NT = 49
NP = NT * TN
BLK = 128
NB = NP // BLK
B_TOT = Q * K
NW = 32
B_PER_W = B_TOT // NW
CH = 400
QB = 256
CAND = K * BLK

NEG_INF = float("-inf")


def _mm_kernel(x_ref, db_ref, s_ref, bm_ref):
    i = pl.program_id(0)
    s = lax.dot_general(
        x_ref[...], db_ref[...], (((1,), (1,)), ((), ())),
        preferred_element_type=jnp.float32,
        precision=lax.Precision.DEFAULT,
    )
    col = i * TN + lax.broadcasted_iota(jnp.int32, (Q, TN), 1)
    s = jnp.where(col < N, s, NEG_INF)
    for b in range(TN // BLK):
        s_ref[b * Q:(b + 1) * Q, :] = s[:, b * BLK:(b + 1) * BLK]
    bm_ref[0] = jnp.concatenate(
        [jnp.max(s[:, b * BLK:(b + 1) * BLK], axis=1, keepdims=True)
         for b in range(TN // BLK)], axis=1)


_k1 = pl.pallas_call(
    _mm_kernel,
    grid=(NT,),
    in_specs=[
        pl.BlockSpec((Q, D), lambda i: (0, 0)),
        pl.BlockSpec((TN, D), lambda i: (i, 0)),
    ],
    out_specs=[
        pl.BlockSpec(((TN // BLK) * Q, BLK), lambda i: (i, 0)),
        pl.BlockSpec((1, Q, TN // BLK), lambda i: (i, 0, 0)),
    ],
    out_shape=[
        jax.ShapeDtypeStruct((Q * NB, BLK), jnp.float32),
        jax.ShapeDtypeStruct((NT, Q, TN // BLK), jnp.float32),
    ],
)


def _sel_kernel(bm_ref, sidx_ref, lidx_ref, bm_s):
    bm_s[...] = bm_ref[...]
    colio = lax.broadcasted_iota(jnp.int32, (Q, NB), 1)
    qrow = lax.broadcasted_iota(jnp.int32, (Q, 1), 0)
    kio = lax.broadcasted_iota(jnp.int32, (Q, K), 1)

    def body(r, _):
        bm = bm_s[...]
        m = jnp.max(bm, axis=1, keepdims=True)
        idx = jnp.min(jnp.where(bm == m, colio, NB), axis=1, keepdims=True)
        sel = kio == r
        lidx_ref[...] = jnp.where(sel, idx, lidx_ref[...])
        srow = (idx >> 4) * ((TN // BLK) * Q) + (idx & (TN // BLK - 1)) * Q + qrow
        sidx_ref[...] = jnp.where(sel, srow, sidx_ref[...])
        bm_s[...] = jnp.where(colio == idx, NEG_INF, bm)
        return 0

    lax.fori_loop(0, K, body, 0)


_k2 = pl.pallas_call(
    _sel_kernel,
    in_specs=[pl.BlockSpec((Q, NB), lambda: (0, 0))],
    out_specs=[
        pl.BlockSpec((Q, K), lambda: (0, 0)),
        pl.BlockSpec((Q, K), lambda: (0, 0)),
    ],
    out_shape=[
        jax.ShapeDtypeStruct((Q, K), jnp.int32),
        jax.ShapeDtypeStruct((Q, K), jnp.int32),
    ],
    scratch_shapes=[pltpu.VMEM((Q, NB), jnp.float32)],
)


def _gather_body(s_hbm, sidx_hbm, lab_hbm, lidx_hbm, outs_hbm, outl_hbm,
                 sidx_v, srows_v, lidx_v, lrows_v, sem1, sem2):
    wid = lax.axis_index("s") * 2 + lax.axis_index("c")
    base = wid * B_PER_W
    for c in range(B_PER_W // CH):
        off = base + c * CH
        pltpu.sync_copy(sidx_hbm.at[pl.ds(off, CH)], sidx_v)
        pltpu.sync_copy(lidx_hbm.at[pl.ds(off, CH)], lidx_v)
        cp1 = pltpu.async_copy(s_hbm.at[sidx_v], srows_v, sem1)
        cp2 = pltpu.async_copy(lab_hbm.at[lidx_v], lrows_v, sem2)
        cp1.wait()
        cp2.wait()
        pltpu.sync_copy(srows_v, outs_hbm.at[pl.ds(off, CH)])
        pltpu.sync_copy(lrows_v, outl_hbm.at[pl.ds(off, CH)])


_k3 = functools.partial(
    pl.kernel,
    out_type=(
        jax.ShapeDtypeStruct((B_TOT, BLK), jnp.float32),
        jax.ShapeDtypeStruct((B_TOT, BLK), jnp.int32),
    ),
    mesh=plsc.VectorSubcoreMesh(core_axis_name="c", subcore_axis_name="s"),
    scratch_types=[
        pltpu.VMEM((CH,), jnp.int32),
        pltpu.VMEM((CH, BLK), jnp.float32),
        pltpu.VMEM((CH,), jnp.int32),
        pltpu.VMEM((CH, BLK), jnp.int32),
        pltpu.SemaphoreType.DMA,
        pltpu.SemaphoreType.DMA,
    ],
)(_gather_body)


def _hist_kernel(cs_ref, cl_ref, out_ref, h_s):
    h_s[...] = jnp.zeros((QB, C), jnp.float32)
    cio = lax.broadcasted_iota(jnp.int32, (QB, C), 1)

    def body(r, _):
        s = cs_ref[...]
        lab = cl_ref[...]
        m = jnp.max(s, axis=1, keepdims=True)
        lbl = jnp.max(jnp.where(s == m, lab, -1), axis=1, keepdims=True)
        h_s[...] = h_s[...] + (cio == lbl).astype(jnp.float32)
        cs_ref[...] = jnp.where((s == m) & (lab == lbl), NEG_INF, s)
        return 0

    lax.fori_loop(0, K, body, 0)
    out_ref[...] = h_s[...] * jnp.float32(1.0 / K)


_k4 = pl.pallas_call(
    _hist_kernel,
    grid=(Q // QB,),
    in_specs=[
        pl.BlockSpec((QB, CAND), lambda i: (i, 0)),
        pl.BlockSpec((QB, CAND), lambda i: (i, 0)),
    ],
    out_specs=pl.BlockSpec((QB, C), lambda i: (i, 0)),
    out_shape=jax.ShapeDtypeStruct((Q, C), jnp.float32),
    scratch_shapes=[
        pltpu.VMEM((QB, C), jnp.float32),
    ],
)


def kernel(x, database, aux_labels):
    s, bm3 = _k1(x, database)
    sidx, lidx = _k2(bm3.transpose(1, 0, 2).reshape(Q, NB))
    labtab = jnp.concatenate(
        [aux_labels, jnp.zeros((NP - N,), jnp.int32)]).reshape(NB, BLK)
    cand_s, cand_l = _k3(
        s, sidx.reshape(B_TOT), labtab, lidx.reshape(B_TOT))
    return _k4(cand_s.reshape(Q, CAND), cand_l.reshape(Q, CAND))


# speedup vs baseline: 3.5149x; 1.0791x over previous
import functools

import jax
import jax.numpy as jnp
from jax import lax
from jax.experimental import pallas as pl
from jax.experimental.pallas import tpu as pltpu
from jax.experimental.pallas import tpu_sc as plsc

Q = 1024
D = 128
N = 100000
C = 1000
K = 50
TN = 2048
NT = 49
NP = NT * TN
BLK = 128
NB = NP // BLK
B_TOT = Q * K
NW = 32
B_PER_W = B_TOT // NW
CH = 400
QB = 256
CAND = K * BLK

NEG_INF = float("-inf")


def _mm_kernel(x_ref, db_ref, s_ref, bm_ref):
    i = pl.program_id(0)
    s = lax.dot_general(
        x_ref[...], db_ref[...], (((1,), (1,)), ((), ())),
        preferred_element_type=jnp.float32,
        precision=lax.Precision.DEFAULT,
    )
    col = i * TN + lax.broadcasted_iota(jnp.int32, (Q, TN), 1)
    s = jnp.where(col < N, s, NEG_INF)
    for b in range(TN // BLK):
        s_ref[b * Q:(b + 1) * Q, :] = s[:, b * BLK:(b + 1) * BLK]
    bm_ref[0] = jnp.concatenate(
        [jnp.max(s[:, b * BLK:(b + 1) * BLK], axis=1, keepdims=True)
         for b in range(TN // BLK)], axis=1)


_k1 = pl.pallas_call(
    _mm_kernel,
    grid=(NT,),
    in_specs=[
        pl.BlockSpec((Q, D), lambda i: (0, 0)),
        pl.BlockSpec((TN, D), lambda i: (i, 0)),
    ],
    out_specs=[
        pl.BlockSpec(((TN // BLK) * Q, BLK), lambda i: (i, 0)),
        pl.BlockSpec((1, Q, TN // BLK), lambda i: (i, 0, 0)),
    ],
    out_shape=[
        jax.ShapeDtypeStruct((Q * NB, BLK), jnp.float32),
        jax.ShapeDtypeStruct((NT, Q, TN // BLK), jnp.float32),
    ],
)


def _sel_kernel(bm_ref, sidx_ref, lidx_ref, bm_s):
    bm_s[...] = bm_ref[...]
    colio = lax.broadcasted_iota(jnp.int32, (Q, NB), 1)
    qrow = lax.broadcasted_iota(jnp.int32, (Q, 1), 0)
    kio = lax.broadcasted_iota(jnp.int32, (Q, K), 1)

    def body(r, _):
        bm = bm_s[...]
        m = jnp.max(bm, axis=1, keepdims=True)
        idx = jnp.min(jnp.where(bm == m, colio, NB), axis=1, keepdims=True)
        sel = kio == r
        lidx_ref[...] = jnp.where(sel, idx, lidx_ref[...])
        srow = (idx >> 4) * ((TN // BLK) * Q) + (idx & (TN // BLK - 1)) * Q + qrow
        sidx_ref[...] = jnp.where(sel, srow, sidx_ref[...])
        bm_s[...] = jnp.where(colio == idx, NEG_INF, bm)
        return 0

    lax.fori_loop(0, K, body, 0)


_k2 = pl.pallas_call(
    _sel_kernel,
    in_specs=[pl.BlockSpec((Q, NB), lambda: (0, 0))],
    out_specs=[
        pl.BlockSpec((Q, K), lambda: (0, 0)),
        pl.BlockSpec((Q, K), lambda: (0, 0)),
    ],
    out_shape=[
        jax.ShapeDtypeStruct((Q, K), jnp.int32),
        jax.ShapeDtypeStruct((Q, K), jnp.int32),
    ],
    scratch_shapes=[pltpu.VMEM((Q, NB), jnp.float32)],
)


def _gather_body(s_hbm, sidx_hbm, lab_hbm, lidx_hbm, outs_hbm, outl_hbm,
                 sidx_v, srows_v, lidx_v, lrows_v, sem1, sem2):
    wid = lax.axis_index("s") * 2 + lax.axis_index("c")
    base = wid * B_PER_W
    for c in range(B_PER_W // CH):
        off = base + c * CH
        pltpu.sync_copy(sidx_hbm.at[pl.ds(off, CH)], sidx_v)
        pltpu.sync_copy(lidx_hbm.at[pl.ds(off, CH)], lidx_v)
        cp1 = pltpu.async_copy(s_hbm.at[sidx_v], srows_v, sem1)
        cp2 = pltpu.async_copy(lab_hbm.at[lidx_v], lrows_v, sem2)
        cp1.wait()
        cp2.wait()
        pltpu.sync_copy(srows_v, outs_hbm.at[pl.ds(off, CH)])
        pltpu.sync_copy(lrows_v, outl_hbm.at[pl.ds(off, CH)])


_k3 = functools.partial(
    pl.kernel,
    out_type=(
        jax.ShapeDtypeStruct((B_TOT, BLK), jnp.float32),
        jax.ShapeDtypeStruct((B_TOT, BLK), jnp.int32),
    ),
    mesh=plsc.VectorSubcoreMesh(core_axis_name="c", subcore_axis_name="s"),
    scratch_types=[
        pltpu.VMEM((CH,), jnp.int32),
        pltpu.VMEM((CH, BLK), jnp.float32),
        pltpu.VMEM((CH,), jnp.int32),
        pltpu.VMEM((CH, BLK), jnp.int32),
        pltpu.SemaphoreType.DMA,
        pltpu.SemaphoreType.DMA,
    ],
)(_gather_body)


def _hist_kernel(cs_ref, cl_ref, out_ref, h_s):
    h_s[...] = jnp.zeros((QB, C), jnp.float32)
    cio = lax.broadcasted_iota(jnp.int32, (QB, C), 1)

    def body(r, _):
        s = cs_ref[...]
        lab = cl_ref[...]
        m1 = jnp.max(s, axis=1, keepdims=True)
        l1 = jnp.max(jnp.where(s == m1, lab, -1), axis=1, keepdims=True)
        s2 = jnp.where((s == m1) & (lab == l1), NEG_INF, s)
        m2 = jnp.max(s2, axis=1, keepdims=True)
        l2 = jnp.max(jnp.where(s2 == m2, lab, -1), axis=1, keepdims=True)
        h_s[...] = (h_s[...] + (cio == l1).astype(jnp.float32)
                    + (cio == l2).astype(jnp.float32))
        cs_ref[...] = jnp.where((s2 == m2) & (lab == l2), NEG_INF, s2)
        return 0

    lax.fori_loop(0, K // 2, body, 0)
    out_ref[...] = h_s[...] * jnp.float32(1.0 / K)


_k4 = pl.pallas_call(
    _hist_kernel,
    grid=(Q // QB,),
    in_specs=[
        pl.BlockSpec((QB, CAND), lambda i: (i, 0)),
        pl.BlockSpec((QB, CAND), lambda i: (i, 0)),
    ],
    out_specs=pl.BlockSpec((QB, C), lambda i: (i, 0)),
    out_shape=jax.ShapeDtypeStruct((Q, C), jnp.float32),
    scratch_shapes=[
        pltpu.VMEM((QB, C), jnp.float32),
    ],
)


def kernel(x, database, aux_labels):
    s, bm3 = _k1(x, database)
    sidx, lidx = _k2(bm3.transpose(1, 0, 2).reshape(Q, NB))
    labtab = jnp.concatenate(
        [aux_labels, jnp.zeros((NP - N,), jnp.int32)]).reshape(NB, BLK)
    cand_s, cand_l = _k3(
        s, sidx.reshape(B_TOT), labtab, lidx.reshape(B_TOT))
    return _k4(cand_s.reshape(Q, CAND), cand_l.reshape(Q, CAND))


# speedup vs baseline: 3.8254x; 1.0883x over previous
import functools

import jax
import jax.numpy as jnp
from jax import lax
from jax.experimental import pallas as pl
from jax.experimental.pallas import tpu as pltpu
from jax.experimental.pallas import tpu_sc as plsc

Q = 1024
D = 128
N = 100000
C = 1000
K = 50
TN = 2048
NT = 49
NP = NT * TN
BLK = 128
NB = NP // BLK
B_TOT = Q * K
NW = 32
B_PER_W = B_TOT // NW
CH = 400
QB = 256
CAND = K * BLK

NEG_INF = float("-inf")


def _mm_kernel(x_ref, db_ref, s_ref, bm_ref):
    i = pl.program_id(0)
    s = lax.dot_general(
        x_ref[...], db_ref[...], (((1,), (1,)), ((), ())),
        preferred_element_type=jnp.float32,
        precision=lax.Precision.DEFAULT,
    )
    col = i * TN + lax.broadcasted_iota(jnp.int32, (Q, TN), 1)
    s = jnp.where(col < N, s, NEG_INF)
    for b in range(TN // BLK):
        s_ref[b * Q:(b + 1) * Q, :] = s[:, b * BLK:(b + 1) * BLK]
    bm_ref[0] = jnp.concatenate(
        [jnp.max(s[:, b * BLK:(b + 1) * BLK], axis=1, keepdims=True)
         for b in range(TN // BLK)], axis=1)


_k1 = pl.pallas_call(
    _mm_kernel,
    grid=(NT,),
    in_specs=[
        pl.BlockSpec((Q, D), lambda i: (0, 0)),
        pl.BlockSpec((TN, D), lambda i: (i, 0)),
    ],
    out_specs=[
        pl.BlockSpec(((TN // BLK) * Q, BLK), lambda i: (i, 0)),
        pl.BlockSpec((1, Q, TN // BLK), lambda i: (i, 0, 0)),
    ],
    out_shape=[
        jax.ShapeDtypeStruct((Q * NB, BLK), jnp.float32),
        jax.ShapeDtypeStruct((NT, Q, TN // BLK), jnp.float32),
    ],
)


def _sel_kernel(bm_ref, sidx_ref, lidx_ref, bm_s):
    bm_s[...] = bm_ref[...]
    colio = lax.broadcasted_iota(jnp.int32, (Q, NB), 1)
    qrow = lax.broadcasted_iota(jnp.int32, (Q, 1), 0)
    kio = lax.broadcasted_iota(jnp.int32, (Q, K), 1)

    def body(r, _):
        bm = bm_s[...]
        lacc = lidx_ref[...]
        sacc = sidx_ref[...]
        for u in range(2):
            m = jnp.max(bm, axis=1, keepdims=True)
            idx = jnp.min(jnp.where(bm == m, colio, NB), axis=1, keepdims=True)
            sel = kio == 2 * r + u
            lacc = jnp.where(sel, idx, lacc)
            srow = ((idx >> 4) * ((TN // BLK) * Q)
                    + (idx & (TN // BLK - 1)) * Q + qrow)
            sacc = jnp.where(sel, srow, sacc)
            bm = jnp.where(colio == idx, NEG_INF, bm)
        lidx_ref[...] = lacc
        sidx_ref[...] = sacc
        bm_s[...] = bm
        return 0

    lax.fori_loop(0, K // 2, body, 0)


_k2 = pl.pallas_call(
    _sel_kernel,
    in_specs=[pl.BlockSpec((Q, NB), lambda: (0, 0))],
    out_specs=[
        pl.BlockSpec((Q, K), lambda: (0, 0)),
        pl.BlockSpec((Q, K), lambda: (0, 0)),
    ],
    out_shape=[
        jax.ShapeDtypeStruct((Q, K), jnp.int32),
        jax.ShapeDtypeStruct((Q, K), jnp.int32),
    ],
    scratch_shapes=[pltpu.VMEM((Q, NB), jnp.float32)],
)


def _gather_body(s_hbm, sidx_hbm, lab_hbm, lidx_hbm, outs_hbm, outl_hbm,
                 sidx_v, srows_v, lidx_v, lrows_v, sem1, sem2):
    wid = lax.axis_index("s") * 2 + lax.axis_index("c")
    base = wid * B_PER_W
    for c in range(B_PER_W // CH):
        off = base + c * CH
        pltpu.sync_copy(sidx_hbm.at[pl.ds(off, CH)], sidx_v)
        pltpu.sync_copy(lidx_hbm.at[pl.ds(off, CH)], lidx_v)
        cp1 = pltpu.async_copy(s_hbm.at[sidx_v], srows_v, sem1)
        cp2 = pltpu.async_copy(lab_hbm.at[lidx_v], lrows_v, sem2)
        cp1.wait()
        cp2.wait()
        pltpu.sync_copy(srows_v, outs_hbm.at[pl.ds(off, CH)])
        pltpu.sync_copy(lrows_v, outl_hbm.at[pl.ds(off, CH)])


_k3 = functools.partial(
    pl.kernel,
    out_type=(
        jax.ShapeDtypeStruct((B_TOT, BLK), jnp.float32),
        jax.ShapeDtypeStruct((B_TOT, BLK), jnp.int32),
    ),
    mesh=plsc.VectorSubcoreMesh(core_axis_name="c", subcore_axis_name="s"),
    scratch_types=[
        pltpu.VMEM((CH,), jnp.int32),
        pltpu.VMEM((CH, BLK), jnp.float32),
        pltpu.VMEM((CH,), jnp.int32),
        pltpu.VMEM((CH, BLK), jnp.int32),
        pltpu.SemaphoreType.DMA,
        pltpu.SemaphoreType.DMA,
    ],
)(_gather_body)


def _hist_kernel(cs_ref, cl_ref, out_ref, h_s):
    h_s[...] = jnp.zeros((QB, C), jnp.float32)
    cio = lax.broadcasted_iota(jnp.int32, (QB, C), 1)

    def body(r, _):
        s = cs_ref[...]
        lab = cl_ref[...]
        hacc = h_s[...]
        for _ in range(5):
            m1 = jnp.max(s, axis=1, keepdims=True)
            l1 = jnp.max(jnp.where(s == m1, lab, -1), axis=1, keepdims=True)
            hacc = hacc + (cio == l1).astype(jnp.float32)
            s = jnp.where((s == m1) & (lab == l1), NEG_INF, s)
        h_s[...] = hacc
        cs_ref[...] = s
        return 0

    lax.fori_loop(0, K // 5, body, 0)
    out_ref[...] = h_s[...] * jnp.float32(1.0 / K)


_k4 = pl.pallas_call(
    _hist_kernel,
    grid=(Q // QB,),
    in_specs=[
        pl.BlockSpec((QB, CAND), lambda i: (i, 0)),
        pl.BlockSpec((QB, CAND), lambda i: (i, 0)),
    ],
    out_specs=pl.BlockSpec((QB, C), lambda i: (i, 0)),
    out_shape=jax.ShapeDtypeStruct((Q, C), jnp.float32),
    scratch_shapes=[
        pltpu.VMEM((QB, C), jnp.float32),
    ],
)


def kernel(x, database, aux_labels):
    s, bm3 = _k1(x, database)
    sidx, lidx = _k2(bm3.transpose(1, 0, 2).reshape(Q, NB))
    labtab = jnp.concatenate(
        [aux_labels, jnp.zeros((NP - N,), jnp.int32)]).reshape(NB, BLK)
    cand_s, cand_l = _k3(
        s, sidx.reshape(B_TOT), labtab, lidx.reshape(B_TOT))
    return _k4(cand_s.reshape(Q, CAND), cand_l.reshape(Q, CAND))
